# Initial kernel scaffold; baseline (speedup 1.0000x reference)
import jax
import jax.numpy as jnp
from jax.experimental import pallas as pl


def kernel(pos, batch, params):
    raise NotImplementedError("write your pallas kernel here")


# speedup vs baseline: 4.6521x; 4.6521x over previous
import functools

import jax
import jax.numpy as jnp
from jax.experimental import pallas as pl

_EPS = 1e-5
_K = 8
_NUM_GRAPHS = 8
_NP = 10240
_BM = 256


def _knn_body(xq_ref, bq_ref, x_ref, bk_ref, xn_ref, out_ref):
    xq = xq_ref[...]
    dots = jax.lax.dot_general(
        xq, x_ref[...], (((1,), (1,)), ((), ())),
        preferred_element_type=jnp.float32)
    d2 = jnp.sum(xq * xq, axis=1, keepdims=True) + xn_ref[...] - 2.0 * dots
    mask = bq_ref[...] == bk_ref[...]
    big = jnp.float32(jnp.inf)
    d2 = jnp.where(mask, d2, big)
    cols = jax.lax.broadcasted_iota(jnp.int32, d2.shape, 1)
    picks = []
    for _ in range(_K):
        m = jnp.min(d2, axis=1, keepdims=True)
        sel = jnp.where(d2 <= m, cols, jnp.int32(2 ** 30))
        j = jnp.min(sel, axis=1)
        picks.append(j)
        d2 = jnp.where(cols == j[:, None], big, d2)
    out_ref[...] = jnp.stack(picks, axis=1)


@functools.partial(jax.jit, static_argnums=())
def _knn_pallas(x, batch):
    n, d = x.shape
    xpad = jnp.zeros((_NP, d), x.dtype).at[:n].set(x)
    bpad = jnp.full((_NP,), -1, jnp.int32).at[:n].set(batch)
    xn = jnp.sum(xpad * xpad, axis=1)[None, :]
    bq = bpad[:, None]
    bk = bpad[None, :]
    idx = pl.pallas_call(
        _knn_body,
        grid=(_NP // _BM,),
        in_specs=[
            pl.BlockSpec((_BM, d), lambda i: (i, 0)),
            pl.BlockSpec((_BM, 1), lambda i: (i, 0)),
            pl.BlockSpec((_NP, d), lambda i: (0, 0)),
            pl.BlockSpec((1, _NP), lambda i: (0, 0)),
            pl.BlockSpec((1, _NP), lambda i: (0, 0)),
        ],
        out_specs=pl.BlockSpec((_BM, _K), lambda i: (i, 0)),
        out_shape=jax.ShapeDtypeStruct((_NP, _K), jnp.int32),
    )(xpad, bq, xpad, bk, xn)
    return idx[:n]


def _mlp(p, x):
    h = x @ p["W"] + p["b"]
    mu = jnp.mean(h, axis=0)
    var = jnp.var(h, axis=0)
    h = (h - mu) / jnp.sqrt(var + _EPS) * p["gamma"] + p["beta"]
    return h * jax.nn.sigmoid(h)


def _conv(p, x, batch):
    n, d = x.shape
    idx = _knn_pallas(x, batch)
    xj = x[idx]
    xi = jnp.broadcast_to(x[:, None, :], (n, _K, d))
    e = jnp.concatenate([xi, xj - xi], axis=-1).reshape(n * _K, 2 * d)
    m = _mlp(p["msg"], e)
    g = _mlp(p["gate"], m).reshape(n, _K)
    w = jax.nn.softmax(g, axis=1)
    return jnp.sum(w[:, :, None] * m.reshape(n, _K, -1), axis=1)


def kernel(pos, batch, params):
    x1 = _conv(params["conv1"], pos, batch)
    x2 = _conv(params["conv2"], x1, batch)
    x3 = _conv(params["conv3"], x2, batch)
    x4 = _conv(params["conv4"], x3, batch)
    x = jnp.concatenate([x1, x2, x3, x4], axis=-1)
    h = _mlp(params["shared"], x)
    gate = _mlp(params["aggr_gate"], h)
    mx = jax.ops.segment_max(gate, batch, num_segments=_NUM_GRAPHS)
    ex = jnp.exp(gate - mx[batch])
    s = jax.ops.segment_sum(ex, batch, num_segments=_NUM_GRAPHS)
    w = ex / s[batch]
    pooled = jax.ops.segment_sum(w * h, batch, num_segments=_NUM_GRAPHS)
    return _mlp(params["out"], pooled)
